# Initial kernel scaffold; baseline (speedup 1.0000x reference)
import jax
import jax.numpy as jnp
from jax.experimental import pallas as pl


def kernel(x, dim):
    raise NotImplementedError("write your pallas kernel here")


# speedup vs baseline: 2.0240x; 2.0240x over previous
import functools

import jax
import jax.numpy as jnp
from jax import lax
from jax.experimental import pallas as pl
from jax.experimental.pallas import tpu as pltpu
from jax.experimental.pallas import tpu_sc as plsc

_LANES = 16


def _sc_cumsum_2d(x2d, batch, seq):
    rows, feat = x2d.shape
    info = plsc.get_sparse_core_info()
    nc, ns = info.num_cores, info.num_subcores
    nw = nc * ns
    strips_per_batch = nw // batch
    fw = feat // strips_per_batch
    assert feat % strips_per_batch == 0 and fw % _LANES == 0
    nvec = fw // _LANES
    ch = 512
    assert seq % ch == 0
    nchunks = seq // ch

    mesh = plsc.VectorSubcoreMesh(core_axis_name="c", subcore_axis_name="s")

    @functools.partial(
        pl.kernel,
        mesh=mesh,
        out_type=jax.ShapeDtypeStruct((rows, feat), jnp.float32),
        scratch_types=[pltpu.VMEM((ch, fw), jnp.float32)],
    )
    def run(x_hbm, o_hbm, buf):
        wid = lax.axis_index("s") * nc + lax.axis_index("c")
        b = wid // strips_per_batch
        f0 = (wid % strips_per_batch) * fw
        row0 = b * seq

        def chunk_body(ci, accs):
            t0 = row0 + ci * ch
            pltpu.sync_copy(x_hbm.at[pl.ds(t0, ch), pl.ds(f0, fw)], buf)

            def row_body(t, accs):
                new = []
                for j in range(nvec):
                    a = accs[j] + buf[t, pl.ds(j * _LANES, _LANES)]
                    buf[t, pl.ds(j * _LANES, _LANES)] = a
                    new.append(a)
                return tuple(new)

            accs = lax.fori_loop(0, ch, row_body, accs)
            pltpu.sync_copy(buf, o_hbm.at[pl.ds(t0, ch), pl.ds(f0, fw)])
            return accs

        zero = jnp.zeros((_LANES,), jnp.float32)
        lax.fori_loop(0, nchunks, chunk_body, (zero,) * nvec)

    return run(x2d)


def kernel(x, dim):
    del dim
    b, s, f = x.shape
    out = _sc_cumsum_2d(x.reshape(b * s, f), b, s)
    return out.reshape(b, s, f)


# speedup vs baseline: 2.7453x; 1.3564x over previous
import functools

import jax
import jax.numpy as jnp
from jax import lax
from jax.experimental import pallas as pl
from jax.experimental.pallas import tpu as pltpu
from jax.experimental.pallas import tpu_sc as plsc

_LANES = 16
_NBUF = 3


def _sc_cumsum_2d(x2d, batch, seq):
    rows, feat = x2d.shape
    info = plsc.get_sparse_core_info()
    nc, ns = info.num_cores, info.num_subcores
    nw = nc * ns
    strips_per_batch = nw // batch
    fw = feat // strips_per_batch
    assert feat % strips_per_batch == 0 and fw % _LANES == 0
    nvec = fw // _LANES
    ch = 256
    assert seq % ch == 0
    nchunks = seq // ch

    mesh = plsc.VectorSubcoreMesh(core_axis_name="c", subcore_axis_name="s")

    @functools.partial(
        pl.kernel,
        mesh=mesh,
        out_type=jax.ShapeDtypeStruct((rows, feat), jnp.float32),
        scratch_types=(
            [pltpu.VMEM((ch, fw), jnp.float32) for _ in range(_NBUF)]
            + [pltpu.SemaphoreType.DMA for _ in range(2 * _NBUF)]
        ),
    )
    def run(x_hbm, o_hbm, *scratch):
        bufs = scratch[:_NBUF]
        in_sems = scratch[_NBUF : 2 * _NBUF]
        out_sems = scratch[2 * _NBUF :]

        wid = lax.axis_index("s") * nc + lax.axis_index("c")
        b = wid // strips_per_batch
        f0 = (wid % strips_per_batch) * fw
        row0 = b * seq

        def src(ci):
            return x_hbm.at[pl.ds(row0 + ci * ch, ch), pl.ds(f0, fw)]

        def dst(ci):
            return o_hbm.at[pl.ds(row0 + ci * ch, ch), pl.ds(f0, fw)]

        def make_row_body(p):
            def row_body(t, accs):
                new = []
                for j in range(nvec):
                    a = accs[j] + bufs[p][t, pl.ds(j * _LANES, _LANES)]
                    bufs[p][t, pl.ds(j * _LANES, _LANES)] = a
                    new.append(a)
                return tuple(new)

            return row_body

        in_handles = [None] * nchunks
        out_handles = [None] * nchunks
        in_handles[0] = pltpu.async_copy(src(0), bufs[0], in_sems[0])
        accs = tuple(jnp.zeros((_LANES,), jnp.float32) for _ in range(nvec))
        for ci in range(nchunks):
            p = ci % _NBUF
            if ci + 1 < nchunks:
                q = (ci + 1) % _NBUF
                if ci - 2 >= 0:
                    out_handles[ci - 2].wait()
                in_handles[ci + 1] = pltpu.async_copy(src(ci + 1), bufs[q], in_sems[q])
            in_handles[ci].wait()
            accs = lax.fori_loop(0, ch, make_row_body(p), accs)
            out_handles[ci] = pltpu.async_copy(bufs[p], dst(ci), out_sems[p])
        out_handles[nchunks - 2].wait()
        out_handles[nchunks - 1].wait()

    return run(x2d)


def kernel(x, dim):
    del dim
    b, s, f = x.shape
    out = _sc_cumsum_2d(x.reshape(b * s, f), b, s)
    return out.reshape(b, s, f)
